# Initial kernel scaffold; baseline (speedup 1.0000x reference)
import jax
import jax.numpy as jnp
from jax.experimental import pallas as pl


def kernel(rpn_cls_prob_reshape, rpn_bbox_pred, im_info):
    raise NotImplementedError("write your pallas kernel here")


# speedup vs baseline: 318.8077x; 318.8077x over previous
import functools

import jax
import jax.numpy as jnp
from jax.experimental import pallas as pl
from jax.experimental.pallas import tpu as pltpu

_A = 9
_H = 50
_W = 76
_N = _H * _W * _A
_ROWS = 272
_NP = _ROWS * 128
_PRE_NMS = 6000
_POST_NMS = 300
_OUT_ROWS = 304
_NMS_T = 0.7
_FEAT_STRIDE = 16.0
_MIN_SIZE = 3.0

_ANCH = (
    (-84.0, -40.0, 99.0, 55.0),
    (-176.0, -88.0, 191.0, 103.0),
    (-360.0, -184.0, 375.0, 199.0),
    (-56.0, -56.0, 71.0, 71.0),
    (-120.0, -120.0, 135.0, 135.0),
    (-248.0, -248.0, 263.0, 263.0),
    (-36.0, -80.0, 51.0, 95.0),
    (-80.0, -168.0, 95.0, 183.0),
    (-168.0, -344.0, 183.0, 359.0),
)


def _anchor_coord(a, coord_idx):
    out = jnp.full(a.shape, _ANCH[0][coord_idx], jnp.float32)
    for j in range(1, _A):
        out = jnp.where(a == j, _ANCH[j][coord_idx], out)
    return out


def _body(scores_in, dx_in, dy_in, dw_in, dh_in, iminfo, out_ref,
          x1_r, y1_r, x2_r, y2_r, ar_r, sl_r):
    _IMIN = jnp.int32(-2147483648)
    im_h = iminfo[0, 0]
    im_w = iminfo[0, 1]
    min_sz = _MIN_SIZE * iminfo[0, 2]

    i2 = (jax.lax.broadcasted_iota(jnp.int32, (_ROWS, 128), 0) * 128
          + jax.lax.broadcasted_iota(jnp.int32, (_ROWS, 128), 1))
    a = i2 % _A
    cell = i2 // _A
    colf = (cell % _W).astype(jnp.float32) * _FEAT_STRIDE
    rowf = (cell // _W).astype(jnp.float32) * _FEAT_STRIDE
    ax1 = _anchor_coord(a, 0) + colf
    ay1 = _anchor_coord(a, 1) + rowf
    ax2 = _anchor_coord(a, 2) + colf
    ay2 = _anchor_coord(a, 3) + rowf
    w = ax2 - ax1 + 1.0
    h = ay2 - ay1 + 1.0
    cx = ax1 + 0.5 * w
    cy = ay1 + 0.5 * h
    pcx = dx_in[...] * w + cx
    pcy = dy_in[...] * h + cy
    pw = jnp.exp(dw_in[...]) * w
    ph = jnp.exp(dh_in[...]) * h
    x1 = jnp.clip(pcx - 0.5 * pw, 0.0, im_w - 1.0)
    y1 = jnp.clip(pcy - 0.5 * ph, 0.0, im_h - 1.0)
    x2 = jnp.clip(pcx + 0.5 * pw, 0.0, im_w - 1.0)
    y2 = jnp.clip(pcy + 0.5 * ph, 0.0, im_h - 1.0)
    ws = x2 - x1 + 1.0
    hs = y2 - y1 + 1.0
    valid = (ws >= min_sz) & (hs >= min_sz) & (i2 < _N)
    x1_r[...] = x1
    y1_r[...] = y1
    x2_r[...] = x2
    y2_r[...] = y2
    ar_r[...] = ws * hs

    scores = scores_in[...]
    ukey = jnp.where(valid, scores.view(jnp.int32) + 1, 0)
    fkey = ukey ^ _IMIN

    def bs_bit(b, cu):
        cand = cu | jax.lax.shift_left(jnp.int32(1), 31 - b)
        cnt = jnp.sum((fkey >= (cand ^ _IMIN)).astype(jnp.int32))
        return jnp.where(cnt >= _PRE_NMS, cand, cu)

    t_bits = jax.lax.fori_loop(0, 32, bs_bit, jnp.int32(0))
    ft = t_bits ^ _IMIN
    slots = _PRE_NMS - jnp.sum((fkey > ft).astype(jnp.int32))
    eqm = (ukey == t_bits).astype(jnp.int32)

    def bs_idx(b, c):
        cand = c | jax.lax.shift_left(jnp.int32(1), 16 - b)
        cnt = jnp.sum(jnp.where(i2 < cand, eqm, 0))
        return jnp.where(cnt <= slots, cand, c)

    c = jax.lax.fori_loop(1, 17, bs_idx, jnp.int32(0))
    elig = valid & ((fkey > ft) | ((ukey == t_bits) & (i2 < c)))
    sl_r[...] = jnp.where(elig, scores, -jnp.inf)

    lane = jax.lax.broadcasted_iota(jnp.int32, (1, 128), 1)

    def nms_step(k, _):
        sl = sl_r[...]
        m = jnp.max(sl)
        found = m > -jnp.inf
        pick = jnp.min(jnp.where(sl == m, i2, _NP))
        pick = jnp.where(pick == _NP, 0, pick)
        pm = (i2 == pick)
        zero = jnp.float32(0.0)
        px1 = jnp.sum(jnp.where(pm, x1_r[...], zero))
        py1 = jnp.sum(jnp.where(pm, y1_r[...], zero))
        px2 = jnp.sum(jnp.where(pm, x2_r[...], zero))
        py2 = jnp.sum(jnp.where(pm, y2_r[...], zero))
        pa = jnp.sum(jnp.where(pm, ar_r[...], zero))
        ww = jnp.maximum(0.0, jnp.minimum(px2, x2_r[...])
                         - jnp.maximum(px1, x1_r[...]) + 1.0)
        hh = jnp.maximum(0.0, jnp.minimum(py2, y2_r[...])
                         - jnp.maximum(py1, y1_r[...]) + 1.0)
        inter = ww * hh
        ovr = inter / (pa + ar_r[...] - inter)
        kill = (ovr > _NMS_T) | pm
        sl_r[...] = jnp.where(found & kill, -jnp.inf, sl)
        row = jnp.zeros((1, 128), jnp.float32)
        row = jnp.where(lane == 1, px1, row)
        row = jnp.where(lane == 2, py1, row)
        row = jnp.where(lane == 3, px2, row)
        row = jnp.where(lane == 4, py2, row)
        prev0 = out_ref[0:1, :]
        row = jnp.where(found | (k == 0), row, prev0)
        out_ref[pl.ds(k, 1), :] = row
        return 0

    jax.lax.fori_loop(0, _POST_NMS, nms_step, 0)


@jax.jit
def kernel(rpn_cls_prob_reshape, rpn_bbox_pred, im_info):
    scores = jnp.transpose(rpn_cls_prob_reshape[:, _A:], (0, 2, 3, 1)).reshape(-1)
    d = jnp.transpose(rpn_bbox_pred, (0, 2, 3, 1)).reshape(-1, 4)
    pad = _NP - _N

    def p2(v, fill=0.0):
        return jnp.pad(v, (0, pad), constant_values=fill).reshape(_ROWS, 128)

    out = pl.pallas_call(
        _body,
        out_shape=jax.ShapeDtypeStruct((_OUT_ROWS, 128), jnp.float32),
        in_specs=[
            pl.BlockSpec(memory_space=pltpu.VMEM),
            pl.BlockSpec(memory_space=pltpu.VMEM),
            pl.BlockSpec(memory_space=pltpu.VMEM),
            pl.BlockSpec(memory_space=pltpu.VMEM),
            pl.BlockSpec(memory_space=pltpu.VMEM),
            pl.BlockSpec(memory_space=pltpu.SMEM),
        ],
        out_specs=pl.BlockSpec(memory_space=pltpu.VMEM),
        scratch_shapes=[pltpu.VMEM((_ROWS, 128), jnp.float32)] * 6,
    )(p2(scores), p2(d[:, 0]), p2(d[:, 1]), p2(d[:, 2]), p2(d[:, 3]), im_info)
    return out[:_POST_NMS, :5]
